# baseline (speedup 1.0000x reference)
import functools

import jax
import jax.numpy as jnp
from jax import lax
from jax.experimental import pallas as pl
from jax.experimental.pallas import tpu as pltpu
from jax.experimental.pallas import tpu_sc as plsc

N = 10000
E = 320000
D = 128
NC, NS = 2, 16
EPS_ = E // NS
CB = 128
NCH_A = 160
NCH_D = NCH_A // NC
NCH_H = NCH_D // 2
EPS_P = NCH_A * CB
NPAD = 10240
TRASH = N + 16
RPT = NPAD // NS
EPS = 1e-5

_mesh = plsc.VectorSubcoreMesh(core_axis_name="c", subcore_axis_name="s")


@functools.partial(
    pl.kernel,
    out_type=(
        jax.ShapeDtypeStruct((NC, NPAD), jnp.float32),
        jax.ShapeDtypeStruct((NC, NPAD), jnp.float32),
    ),
    mesh=_mesh,
    scratch_types=[
        pltpu.VMEM((NCH_D, CB), jnp.int32),
        pltpu.VMEM((NCH_D, CB), jnp.int32),
        pltpu.VMEM((CB,), jnp.float32),
        pltpu.VMEM_SHARED((NPAD,), jnp.float32),
        pltpu.VMEM_SHARED((NPAD,), jnp.float32),
    ],
)
def _deg_kernel(src_hbm, dst_hbm, zeros_hbm, odeg_hbm, ideg_hbm,
                sidx, didx, ones_v, osh, ish):
    c = lax.axis_index("c")
    s = lax.axis_index("s")
    pltpu.sync_copy(zeros_hbm.at[pl.ds(s * RPT, RPT)], osh.at[pl.ds(s * RPT, RPT)])
    pltpu.sync_copy(zeros_hbm.at[pl.ds(s * RPT, RPT)], ish.at[pl.ds(s * RPT, RPT)])
    pltpu.sync_copy(src_hbm.at[s, pl.ds(c * NCH_D, NCH_D)], sidx)
    pltpu.sync_copy(dst_hbm.at[s, pl.ds(c * NCH_D, NCH_D)], didx)
    for i in range(CB // 16):
        ones_v[pl.ds(i * 16, 16)] = jnp.full((16,), 1.0, jnp.float32)
    plsc.subcore_barrier()

    def body(j, carry):
        pltpu.sync_copy(ones_v, osh.at[sidx.at[j]], add=True)
        pltpu.sync_copy(ones_v, ish.at[didx.at[j]], add=True)
        return carry

    lax.fori_loop(0, NCH_D, body, 0)
    plsc.subcore_barrier()
    pltpu.sync_copy(osh.at[pl.ds(s * RPT, RPT)], odeg_hbm.at[c, pl.ds(s * RPT, RPT)])
    pltpu.sync_copy(ish.at[pl.ds(s * RPT, RPT)], ideg_hbm.at[c, pl.ds(s * RPT, RPT)])


@functools.partial(
    pl.kernel,
    out_type=jax.ShapeDtypeStruct((NC, NPAD, D), jnp.float32),
    mesh=_mesh,
    scratch_types=[
        pltpu.VMEM((NCH_D, CB), jnp.int32),
        pltpu.VMEM((NCH_D, CB), jnp.int32),
        pltpu.VMEM((CB, D), jnp.float32),
        pltpu.VMEM_SHARED((NPAD, D), jnp.float32),
    ],
)
def _agg_kernel(src_hbm, dst_hbm, feat_hbm, zeros_hbm, out_hbm,
                sidx, didx, buf, agg_sh):
    c = lax.axis_index("c")
    s = lax.axis_index("s")
    pltpu.sync_copy(zeros_hbm.at[pl.ds(s * RPT, RPT)],
                    agg_sh.at[pl.ds(s * RPT, RPT)])
    pltpu.sync_copy(src_hbm.at[s, pl.ds(c * NCH_D, NCH_D)], sidx)
    pltpu.sync_copy(dst_hbm.at[s, pl.ds(c * NCH_D, NCH_D)], didx)
    plsc.subcore_barrier()

    def body(j, carry):
        pltpu.sync_copy(feat_hbm.at[sidx.at[j]], buf)
        pltpu.sync_copy(buf, agg_sh.at[didx.at[j]], add=True)
        return carry

    lax.fori_loop(0, NCH_D, body, 0)
    plsc.subcore_barrier()
    pltpu.sync_copy(agg_sh.at[pl.ds(s * RPT, RPT)],
                    out_hbm.at[c, pl.ds(s * RPT, RPT)])


def _scale_body(x_ref, od0_ref, od1_ref, id0_ref, id1_ref, feat_ref, invin_ref):
    odeg = jnp.maximum(od0_ref[...] + od1_ref[...], 1.0)
    ideg = jnp.maximum(id0_ref[...] + id1_ref[...], 1.0)
    feat_ref[0:N, :] = x_ref[...] * lax.rsqrt(odeg)
    feat_ref[N:NPAD, :] = jnp.zeros((NPAD - N, D), jnp.float32)
    invin_ref[...] = lax.rsqrt(ideg)


_scale_call = pl.pallas_call(
    _scale_body,
    out_shape=(
        jax.ShapeDtypeStruct((NPAD, D), jnp.float32),
        jax.ShapeDtypeStruct((N, 1), jnp.float32),
    ),
)


def _final_body(p_ref, invin_ref, x_ref, w_ref, wres_ref, g_ref, b_ref, out_ref):
    agg = (p_ref[0, :N, :] + p_ref[1, :N, :]) * invin_ref[...]
    gX = jnp.maximum(jnp.dot(agg, w_ref[...], preferred_element_type=jnp.float32), 0.0)
    res = jnp.maximum(jnp.dot(x_ref[...], wres_ref[...], preferred_element_type=jnp.float32), 0.0)
    h = gX + res
    mean = jnp.mean(h, axis=0, keepdims=True)
    hm = h - mean
    var = jnp.mean(hm * hm, axis=0, keepdims=True)
    out_ref[...] = hm * lax.rsqrt(var + EPS) * g_ref[...] + b_ref[...]


_final_call = pl.pallas_call(
    _final_body,
    out_shape=jax.ShapeDtypeStruct((N, D), jnp.float32),
)


def kernel(X, edge_index, W, W_res, gamma, beta):
    src = edge_index[0].astype(jnp.int32).reshape(NS, EPS_)
    dst = edge_index[1].astype(jnp.int32).reshape(NS, EPS_)
    padc = jnp.full((NS, EPS_P - EPS_), TRASH, jnp.int32)
    src_p = jnp.concatenate([src, padc], axis=1).reshape(NS, NCH_A, CB)
    dst_p = jnp.concatenate([dst, padc], axis=1).reshape(NS, NCH_A, CB)

    z1 = jnp.zeros((NPAD,), jnp.float32)
    odeg_p, ideg_p = _deg_kernel(src_p, dst_p, z1)

    od0 = odeg_p[0, :N].reshape(N, 1)
    od1 = odeg_p[1, :N].reshape(N, 1)
    id0 = ideg_p[0, :N].reshape(N, 1)
    id1 = ideg_p[1, :N].reshape(N, 1)
    feat, inv_in = _scale_call(X, od0, od1, id0, id1)

    z2 = jnp.zeros((NPAD, D), jnp.float32)
    p = _agg_kernel(src_p, dst_p, feat, z2)

    return _final_call(p, inv_in, X, W, W_res,
                       gamma.reshape(1, D), beta.reshape(1, D))


# speedup vs baseline: 2.0753x; 2.0753x over previous
import functools

import jax
import jax.numpy as jnp
from jax import lax
from jax.experimental import pallas as pl
from jax.experimental.pallas import tpu as pltpu
from jax.experimental.pallas import tpu_sc as plsc

N = 10000
E = 320000
D = 128
NC, NS = 2, 16
EPS_ = E // NS
CB = 128
NCH_A = 160
NCH_D = NCH_A // NC
NCH_H = NCH_D // 2
EPS_P = NCH_A * CB
NPAD = 10240
TRASH = N + 16
RPT = NPAD // NS
EPS = 1e-5

_mesh = plsc.VectorSubcoreMesh(core_axis_name="c", subcore_axis_name="s")


@functools.partial(
    pl.kernel,
    out_type=(
        jax.ShapeDtypeStruct((NC, NPAD), jnp.float32),
        jax.ShapeDtypeStruct((NC, NPAD), jnp.float32),
    ),
    mesh=_mesh,
    scratch_types=[
        pltpu.VMEM((NCH_D, CB), jnp.int32),
        pltpu.VMEM((NCH_D, CB), jnp.int32),
        pltpu.VMEM((CB,), jnp.float32),
        pltpu.VMEM_SHARED((NPAD,), jnp.float32),
        pltpu.VMEM_SHARED((NPAD,), jnp.float32),
    ],
)
def _deg_kernel(src_hbm, dst_hbm, zeros_hbm, odeg_hbm, ideg_hbm,
                sidx, didx, ones_v, osh, ish):
    c = lax.axis_index("c")
    s = lax.axis_index("s")
    pltpu.sync_copy(zeros_hbm.at[pl.ds(s * RPT, RPT)], osh.at[pl.ds(s * RPT, RPT)])
    pltpu.sync_copy(zeros_hbm.at[pl.ds(s * RPT, RPT)], ish.at[pl.ds(s * RPT, RPT)])
    pltpu.sync_copy(src_hbm.at[s, pl.ds(c * NCH_D, NCH_D)], sidx)
    pltpu.sync_copy(dst_hbm.at[s, pl.ds(c * NCH_D, NCH_D)], didx)
    for i in range(CB // 16):
        ones_v[pl.ds(i * 16, 16)] = jnp.full((16,), 1.0, jnp.float32)
    plsc.subcore_barrier()

    def body(j, carry):
        pltpu.sync_copy(ones_v, osh.at[sidx.at[j]], add=True)
        pltpu.sync_copy(ones_v, ish.at[didx.at[j]], add=True)
        return carry

    lax.fori_loop(0, NCH_D, body, 0)
    plsc.subcore_barrier()
    pltpu.sync_copy(osh.at[pl.ds(s * RPT, RPT)], odeg_hbm.at[c, pl.ds(s * RPT, RPT)])
    pltpu.sync_copy(ish.at[pl.ds(s * RPT, RPT)], ideg_hbm.at[c, pl.ds(s * RPT, RPT)])


@functools.partial(
    pl.kernel,
    out_type=jax.ShapeDtypeStruct((NC, NPAD, D), jnp.float32),
    mesh=_mesh,
    scratch_types=[
        pltpu.VMEM((NCH_D, CB), jnp.int32),
        pltpu.VMEM((NCH_D, CB), jnp.int32),
        pltpu.VMEM((CB, D), jnp.float32),
        pltpu.VMEM_SHARED((NPAD, D), jnp.float32),
    ],
)
def _agg_kernel(src_hbm, dst_hbm, feat_hbm, zeros_hbm, out_hbm,
                sidx, didx, buf, agg_sh):
    c = lax.axis_index("c")
    s = lax.axis_index("s")
    pltpu.sync_copy(zeros_hbm.at[pl.ds(s * RPT, RPT)],
                    agg_sh.at[pl.ds(s * RPT, RPT)])
    pltpu.sync_copy(src_hbm.at[s, pl.ds(c * NCH_D, NCH_D)], sidx)
    pltpu.sync_copy(dst_hbm.at[s, pl.ds(c * NCH_D, NCH_D)], didx)
    plsc.subcore_barrier()

    def body(j, carry):
        pltpu.sync_copy(feat_hbm.at[sidx.at[j]], buf)
        pltpu.sync_copy(buf, agg_sh.at[didx.at[j]], add=True)
        return carry

    lax.fori_loop(0, NCH_D, body, 0)
    plsc.subcore_barrier()
    pltpu.sync_copy(agg_sh.at[pl.ds(s * RPT, RPT)],
                    out_hbm.at[c, pl.ds(s * RPT, RPT)])


def _scale_body(x_ref, od0_ref, od1_ref, id0_ref, id1_ref, feat_ref, invin_ref):
    odeg = jnp.maximum(od0_ref[...] + od1_ref[...], 1.0)
    ideg = jnp.maximum(id0_ref[...] + id1_ref[...], 1.0)
    feat_ref[0:N, :] = x_ref[...] * lax.rsqrt(odeg)
    feat_ref[N:NPAD, :] = jnp.zeros((NPAD - N, D), jnp.float32)
    invin_ref[...] = lax.rsqrt(ideg)


_scale_call = pl.pallas_call(
    _scale_body,
    out_shape=(
        jax.ShapeDtypeStruct((NPAD, D), jnp.float32),
        jax.ShapeDtypeStruct((N, 1), jnp.float32),
    ),
)


def _final_body(p_ref, invin_ref, x_ref, w_ref, wres_ref, g_ref, b_ref, out_ref):
    agg = (p_ref[0, :N, :] + p_ref[1, :N, :]) * invin_ref[...]
    gX = jnp.maximum(jnp.dot(agg, w_ref[...], preferred_element_type=jnp.float32), 0.0)
    res = jnp.maximum(jnp.dot(x_ref[...], wres_ref[...], preferred_element_type=jnp.float32), 0.0)
    h = gX + res
    mean = jnp.mean(h, axis=0, keepdims=True)
    hm = h - mean
    var = jnp.mean(hm * hm, axis=0, keepdims=True)
    out_ref[...] = hm * lax.rsqrt(var + EPS) * g_ref[...] + b_ref[...]


_final_call = pl.pallas_call(
    _final_body,
    out_shape=jax.ShapeDtypeStruct((N, D), jnp.float32),
)


def kernel(X, edge_index, W, W_res, gamma, beta):
    src = edge_index[0].astype(jnp.int32).reshape(NS, EPS_)
    dst = edge_index[1].astype(jnp.int32).reshape(NS, EPS_)
    pad_pos = jnp.arange(EPS_P - EPS_, dtype=jnp.int32)
    padc = jnp.tile((TRASH + pad_pos % 208)[None, :], (NS, 1))
    src_p = jnp.concatenate([src, padc], axis=1).reshape(NS, NCH_A, CB)
    dst_p = jnp.concatenate([dst, padc], axis=1).reshape(NS, NCH_A, CB)

    z1 = jnp.zeros((NPAD,), jnp.float32)
    odeg_p, ideg_p = _deg_kernel(src_p, dst_p, z1)

    od0 = odeg_p[0, :N].reshape(N, 1)
    od1 = odeg_p[1, :N].reshape(N, 1)
    id0 = ideg_p[0, :N].reshape(N, 1)
    id1 = ideg_p[1, :N].reshape(N, 1)
    feat, inv_in = _scale_call(X, od0, od1, id0, id1)

    z2 = jnp.zeros((NPAD, D), jnp.float32)
    p = _agg_kernel(src_p, dst_p, feat, z2)

    return _final_call(p, inv_in, X, W, W_res,
                       gamma.reshape(1, D), beta.reshape(1, D))


# speedup vs baseline: 2.7295x; 1.3152x over previous
import functools

import jax
import jax.numpy as jnp
from jax import lax
from jax.experimental import pallas as pl
from jax.experimental.pallas import tpu as pltpu
from jax.experimental.pallas import tpu_sc as plsc

N = 10000
E = 320000
D = 128
NC, NS = 2, 16
EPS_ = E // NS
CB = 128
NCH_A = 160
NCH_D = NCH_A // NC
NCH_H = NCH_D // 2
EPS_P = NCH_A * CB
NPAD = 10240
TRASH = N + 16
RPT = NPAD // NS
EPS = 1e-5

_mesh = plsc.VectorSubcoreMesh(core_axis_name="c", subcore_axis_name="s")


@functools.partial(
    pl.kernel,
    out_type=(
        jax.ShapeDtypeStruct((NC, NPAD), jnp.float32),
        jax.ShapeDtypeStruct((NC, NPAD), jnp.float32),
    ),
    mesh=_mesh,
    scratch_types=[
        pltpu.VMEM((NCH_D, CB), jnp.int32),
        pltpu.VMEM((NCH_D, CB), jnp.int32),
        pltpu.VMEM((CB,), jnp.float32),
        pltpu.VMEM_SHARED((NPAD,), jnp.float32),
        pltpu.VMEM_SHARED((NPAD,), jnp.float32),
    ],
)
def _deg_kernel(src_hbm, dst_hbm, zeros_hbm, odeg_hbm, ideg_hbm,
                sidx, didx, ones_v, osh, ish):
    c = lax.axis_index("c")
    s = lax.axis_index("s")
    pltpu.sync_copy(zeros_hbm.at[pl.ds(s * RPT, RPT)], osh.at[pl.ds(s * RPT, RPT)])
    pltpu.sync_copy(zeros_hbm.at[pl.ds(s * RPT, RPT)], ish.at[pl.ds(s * RPT, RPT)])
    pltpu.sync_copy(src_hbm.at[s, pl.ds(c * NCH_D, NCH_D)], sidx)
    pltpu.sync_copy(dst_hbm.at[s, pl.ds(c * NCH_D, NCH_D)], didx)
    for i in range(CB // 16):
        ones_v[pl.ds(i * 16, 16)] = jnp.full((16,), 1.0, jnp.float32)
    plsc.subcore_barrier()

    def body(j, carry):
        pltpu.sync_copy(ones_v, osh.at[sidx.at[j]], add=True)
        pltpu.sync_copy(ones_v, ish.at[didx.at[j]], add=True)
        return carry

    lax.fori_loop(0, NCH_D, body, 0)
    plsc.subcore_barrier()
    pltpu.sync_copy(osh.at[pl.ds(s * RPT, RPT)], odeg_hbm.at[c, pl.ds(s * RPT, RPT)])
    pltpu.sync_copy(ish.at[pl.ds(s * RPT, RPT)], ideg_hbm.at[c, pl.ds(s * RPT, RPT)])


@functools.partial(
    pl.kernel,
    out_type=jax.ShapeDtypeStruct((NC, NPAD, D), jnp.float32),
    mesh=_mesh,
    scratch_types=[
        pltpu.VMEM((NCH_H, CB), jnp.int32),
        pltpu.VMEM((NCH_H, CB), jnp.int32),
        pltpu.VMEM((CB, D), jnp.float32),
        pltpu.VMEM((CB, D), jnp.float32),
        pltpu.VMEM_SHARED((NPAD, D), jnp.float32),
        pltpu.SemaphoreType.DMA,
        pltpu.SemaphoreType.DMA,
    ],
)
def _agg_kernel(src_hbm, dst_hbm, feat_hbm, zeros_hbm, out_hbm,
                sidx, didx, buf_a, buf_b, agg_sh, sem_a, sem_b):
    c = lax.axis_index("c")
    s = lax.axis_index("s")
    pltpu.sync_copy(zeros_hbm.at[pl.ds(s * RPT, RPT)],
                    agg_sh.at[pl.ds(s * RPT, RPT)])
    plsc.subcore_barrier()

    for p in range(2):
        base = c * NCH_D + p * NCH_H
        pltpu.sync_copy(src_hbm.at[s, pl.ds(base, NCH_H)], sidx)
        pltpu.sync_copy(dst_hbm.at[s, pl.ds(base, NCH_H)], didx)
        pltpu.async_copy(feat_hbm.at[sidx.at[0]], buf_a, sem_a)

        def body(g, carry):
            j0 = 2 * g
            pltpu.async_copy(feat_hbm.at[sidx.at[j0 + 1]], buf_b, sem_b)
            pltpu.make_async_copy(feat_hbm.at[sidx.at[j0]], buf_a, sem_a).wait()
            pltpu.sync_copy(buf_a, agg_sh.at[didx.at[j0]], add=True)

            @pl.when(g + 1 < NCH_H // 2)
            def _():
                pltpu.async_copy(feat_hbm.at[sidx.at[j0 + 2]], buf_a, sem_a)

            pltpu.make_async_copy(feat_hbm.at[sidx.at[j0 + 1]], buf_b, sem_b).wait()
            pltpu.sync_copy(buf_b, agg_sh.at[didx.at[j0 + 1]], add=True)
            return carry

        lax.fori_loop(0, NCH_H // 2, body, 0)
    plsc.subcore_barrier()
    pltpu.sync_copy(agg_sh.at[pl.ds(s * RPT, RPT)],
                    out_hbm.at[c, pl.ds(s * RPT, RPT)])


def _scale_body(x_ref, od0_ref, od1_ref, id0_ref, id1_ref, feat_ref, invin_ref):
    odeg = jnp.maximum(od0_ref[...] + od1_ref[...], 1.0)
    ideg = jnp.maximum(id0_ref[...] + id1_ref[...], 1.0)
    feat_ref[0:N, :] = x_ref[...] * lax.rsqrt(odeg)
    feat_ref[N:NPAD, :] = jnp.zeros((NPAD - N, D), jnp.float32)
    invin_ref[...] = lax.rsqrt(ideg)


_scale_call = pl.pallas_call(
    _scale_body,
    out_shape=(
        jax.ShapeDtypeStruct((NPAD, D), jnp.float32),
        jax.ShapeDtypeStruct((N, 1), jnp.float32),
    ),
)


def _final_body(p_ref, invin_ref, x_ref, w_ref, wres_ref, g_ref, b_ref, out_ref):
    agg = (p_ref[0, :N, :] + p_ref[1, :N, :]) * invin_ref[...]
    gX = jnp.maximum(jnp.dot(agg, w_ref[...], preferred_element_type=jnp.float32), 0.0)
    res = jnp.maximum(jnp.dot(x_ref[...], wres_ref[...], preferred_element_type=jnp.float32), 0.0)
    h = gX + res
    mean = jnp.mean(h, axis=0, keepdims=True)
    hm = h - mean
    var = jnp.mean(hm * hm, axis=0, keepdims=True)
    out_ref[...] = hm * lax.rsqrt(var + EPS) * g_ref[...] + b_ref[...]


_final_call = pl.pallas_call(
    _final_body,
    out_shape=jax.ShapeDtypeStruct((N, D), jnp.float32),
)


def kernel(X, edge_index, W, W_res, gamma, beta):
    src = edge_index[0].astype(jnp.int32).reshape(NS, EPS_)
    dst = edge_index[1].astype(jnp.int32).reshape(NS, EPS_)
    pad_pos = jnp.arange(EPS_P - EPS_, dtype=jnp.int32)
    padc = jnp.tile((TRASH + pad_pos % 208)[None, :], (NS, 1))
    src_p = jnp.concatenate([src, padc], axis=1).reshape(NS, NCH_A, CB)
    dst_p = jnp.concatenate([dst, padc], axis=1).reshape(NS, NCH_A, CB)

    z1 = jnp.zeros((NPAD,), jnp.float32)
    odeg_p, ideg_p = _deg_kernel(src_p, dst_p, z1)

    od0 = odeg_p[0, :N].reshape(N, 1)
    od1 = odeg_p[1, :N].reshape(N, 1)
    id0 = ideg_p[0, :N].reshape(N, 1)
    id1 = ideg_p[1, :N].reshape(N, 1)
    feat, inv_in = _scale_call(X, od0, od1, id0, id1)

    z2 = jnp.zeros((NPAD, D), jnp.float32)
    p = _agg_kernel(src_p, dst_p, feat, z2)

    return _final_call(p, inv_in, X, W, W_res,
                       gamma.reshape(1, D), beta.reshape(1, D))
